# baseline (speedup 1.0000x reference)
import jax
import jax.numpy as jnp
from jax import lax
from jax.experimental import pallas as pl
from jax.experimental.pallas import tpu as pltpu
from jax.experimental.pallas import tpu_sc as plsc

HIDDEN = 768
TOKENS = 8192
NUM_TILES = 32
NSPLIT = 2
STOK = TOKENS // NSPLIT
TOK_PER_TILE = STOK // NUM_TILES
CHUNK = 64
EPS = 1e-12
SEQ = 2048
BLK = 512
NBLK = TOKENS // BLK


NRING = 3
GCHUNK = 32
NGCHUNK = TOK_PER_TILE // GCHUNK


def _gather_body(ids_h, wemb_h, out_h,
                 idx_all, buf0, buf1, buf2, sg0, sg1, sg2, ss0, ss1, ss2):
    c = lax.axis_index("c")
    s = lax.axis_index("s")
    wid = s * 2 + c
    base = pl.multiple_of(wid * TOK_PER_TILE, TOK_PER_TILE)

    buf = (buf0, buf1, buf2)
    sg = (sg0, sg1, sg2)
    ss = (ss0, ss1, ss2)

    pltpu.sync_copy(ids_h.at[pl.ds(base, TOK_PER_TILE)], idx_all)

    def fire(ck):
        r = ck % NRING
        return pltpu.async_copy(
            wemb_h.at[idx_all.at[pl.ds(ck * GCHUNK, GCHUNK)]], buf[r], sg[r])

    def scat(ck):
        r = ck % NRING
        off = pl.multiple_of(base + ck * GCHUNK, GCHUNK)
        return pltpu.async_copy(buf[r], out_h.at[pl.ds(off, GCHUNK)], ss[r])

    g_pend = [None] * NGCHUNK
    s_pend = [None] * NGCHUNK
    g_pend[0] = fire(0)
    g_pend[1] = fire(1)
    for ck in range(NGCHUNK):
        if 2 <= ck + 1 < NGCHUNK:
            if ck - 2 >= 0:
                s_pend[ck - 2].wait()
            g_pend[ck + 1] = fire(ck + 1)
        g_pend[ck].wait()
        s_pend[ck] = scat(ck)
    s_pend[NGCHUNK - 2].wait()
    s_pend[NGCHUNK - 1].wait()


def _sc_gather(ids, word_emb):
    run = pl.kernel(
        _gather_body,
        out_type=jax.ShapeDtypeStruct((STOK, HIDDEN), jnp.float32),
        scratch_types=[
            pltpu.VMEM((TOK_PER_TILE,), jnp.int32),
            pltpu.VMEM((GCHUNK, HIDDEN), jnp.float32),
            pltpu.VMEM((GCHUNK, HIDDEN), jnp.float32),
            pltpu.VMEM((GCHUNK, HIDDEN), jnp.float32),
            pltpu.SemaphoreType.DMA(()),
            pltpu.SemaphoreType.DMA(()),
            pltpu.SemaphoreType.DMA(()),
            pltpu.SemaphoreType.DMA(()),
            pltpu.SemaphoreType.DMA(()),
            pltpu.SemaphoreType.DMA(()),
        ],
        mesh=plsc.VectorSubcoreMesh(core_axis_name="c", subcore_axis_name="s"),
        compiler_params=pltpu.CompilerParams(needs_layout_passes=False),
    )
    return run(ids, word_emb)


def _ln_body_first(g_ref, p_ref, tt_ref, te_ref, gm_ref, bt_ref, o_ref):
    _ln_common(g_ref, p_ref, tt_ref, te_ref, gm_ref, bt_ref, o_ref)


def _ln_body_chained(prev_ref, g_ref, p_ref, tt_ref, te_ref, gm_ref, bt_ref,
                     o_ref):
    del prev_ref
    _ln_common(g_ref, p_ref, tt_ref, te_ref, gm_ref, bt_ref, o_ref)


def _ln_common(g_ref, p_ref, tt_ref, te_ref, gm_ref, bt_ref, o_ref):
    tt = tt_ref[...]
    t0 = te_ref[0:1, :][None]
    t1 = te_ref[1:2, :][None]
    x = g_ref[...] + p_ref[...][None] + t0 + tt * (t1 - t0)
    mean = jnp.mean(x, axis=-1, keepdims=True)
    cx = x - mean
    var = jnp.mean(cx * cx, axis=-1, keepdims=True)
    rstd = lax.rsqrt(var + EPS)
    o_ref[...] = cx * rstd * gm_ref[...][None] + bt_ref[...][None]


def _tc_layernorm_slice(prev_out, k, gathered, pos_emb, ttf, type_emb,
                        gamma, beta, bsz):
    half_blks = SEQ // NSPLIT // BLK
    data_specs = [
        pl.BlockSpec((bsz, BLK, HIDDEN), lambda i: (0, i, 0)),
        pl.BlockSpec((BLK, HIDDEN), lambda i: (k * half_blks + i, 0)),
        pl.BlockSpec((bsz, BLK, 1), lambda i: (0, i, 0)),
        pl.BlockSpec((2, HIDDEN), lambda i: (0, 0)),
        pl.BlockSpec((1, HIDDEN), lambda i: (0, 0)),
        pl.BlockSpec((1, HIDDEN), lambda i: (0, 0)),
    ]
    out_spec = pl.BlockSpec((bsz, BLK, HIDDEN),
                            lambda i: (0, k * half_blks + i, 0))
    out_shape = jax.ShapeDtypeStruct((bsz, SEQ, HIDDEN), jnp.float32)
    args = (gathered, pos_emb, ttf, type_emb, gamma, beta)
    if prev_out is None:
        return pl.pallas_call(
            _ln_body_first,
            grid=(half_blks,),
            in_specs=data_specs,
            out_specs=out_spec,
            out_shape=out_shape,
        )(*args)
    return pl.pallas_call(
        _ln_body_chained,
        grid=(half_blks,),
        in_specs=[pl.BlockSpec(memory_space=pl.ANY)] + data_specs,
        out_specs=out_spec,
        out_shape=out_shape,
        input_output_aliases={0: 0},
    )(prev_out, *args)


@jax.jit
def kernel(input_ids, token_type_ids, word_emb, pos_emb, type_emb, gamma, beta):
    bsz, seq = input_ids.shape
    half = seq // NSPLIT
    ids = input_ids.astype(jnp.int32)
    ttf = token_type_ids.reshape(bsz, seq, 1).astype(jnp.float32)
    gm = gamma.reshape(1, HIDDEN)
    bt = beta.reshape(1, HIDDEN)
    gathered = [
        _sc_gather(ids[:, k * half:(k + 1) * half].reshape(-1), word_emb)
        .reshape(bsz, half, HIDDEN)
        for k in range(NSPLIT)
    ]
    out = None
    for k in range(NSPLIT):
        out = _tc_layernorm_slice(
            out, k, gathered[k], pos_emb,
            ttf[:, k * half:(k + 1) * half], type_emb, gm, bt, bsz)
    return out


# speedup vs baseline: 1.0181x; 1.0181x over previous
import jax
import jax.numpy as jnp
from jax import lax
from jax.experimental import pallas as pl
from jax.experimental.pallas import tpu as pltpu
from jax.experimental.pallas import tpu_sc as plsc

HIDDEN = 768
TOKENS = 8192
NUM_TILES = 32
TOK_PER_TILE = TOKENS // NUM_TILES
CHUNK = 64
EPS = 1e-12
SEQ = 2048
BLK = 512
NBLK = TOKENS // BLK


def _gather_body(ids_h, wemb_h, out_h,
                 idx0, idx1, buf0, buf1, sg0, sg1, ss0, ss1):
    c = lax.axis_index("c")
    s = lax.axis_index("s")
    wid = s * 2 + c
    base = pl.multiple_of(wid * TOK_PER_TILE, TOK_PER_TILE)

    idx = (idx0, idx1)
    buf = (buf0, buf1)
    sg = (sg0, sg1)
    ss = (ss0, ss1)

    pltpu.sync_copy(ids_h.at[pl.ds(base, CHUNK)], idx0)
    g0 = pltpu.async_copy(wemb_h.at[idx0], buf0, sg0)
    pltpu.sync_copy(ids_h.at[pl.ds(base + CHUNK, CHUNK)], idx1)
    g1 = pltpu.async_copy(wemb_h.at[idx1], buf1, sg1)
    scat = [None, None]
    g = [g0, g1]
    for ck in range(4):
        b = ck % 2
        off = pl.multiple_of(base + ck * CHUNK, CHUNK)
        g[b].wait()
        scat[b] = pltpu.async_copy(buf[b], out_h.at[pl.ds(off, CHUNK)], ss[b])
        nxt = ck + 2
        if nxt < 4:
            noff = pl.multiple_of(base + nxt * CHUNK, CHUNK)
            scat[b].wait()
            pltpu.sync_copy(ids_h.at[pl.ds(noff, CHUNK)], idx[b])
            g[b] = pltpu.async_copy(wemb_h.at[idx[b]], buf[b], sg[b])
    scat[0].wait()
    scat[1].wait()


def _sc_gather(ids, word_emb):
    run = pl.kernel(
        _gather_body,
        out_type=jax.ShapeDtypeStruct((TOKENS, HIDDEN), jnp.float32),
        scratch_types=[
            pltpu.VMEM((CHUNK,), jnp.int32),
            pltpu.VMEM((CHUNK,), jnp.int32),
            pltpu.VMEM((CHUNK, HIDDEN), jnp.float32),
            pltpu.VMEM((CHUNK, HIDDEN), jnp.float32),
            pltpu.SemaphoreType.DMA(()),
            pltpu.SemaphoreType.DMA(()),
            pltpu.SemaphoreType.DMA(()),
            pltpu.SemaphoreType.DMA(()),
        ],
        mesh=plsc.VectorSubcoreMesh(core_axis_name="c", subcore_axis_name="s"),
        compiler_params=pltpu.CompilerParams(needs_layout_passes=False),
    )
    return run(ids, word_emb)


def _ln_body(g_ref, p_ref, tt_ref, te_ref, gm_ref, bt_ref, o_ref):
    tt = tt_ref[...]
    t0 = te_ref[0:1, :][None]
    t1 = te_ref[1:2, :][None]
    x = g_ref[...] + p_ref[...][None] + t0 + tt * (t1 - t0)
    mean = jnp.mean(x, axis=-1, keepdims=True)
    cx = x - mean
    var = jnp.mean(cx * cx, axis=-1, keepdims=True)
    rstd = lax.rsqrt(var + EPS)
    o_ref[...] = cx * rstd * gm_ref[...][None] + bt_ref[...][None]


def _tc_layernorm(gathered, pos_emb, ttf, type_emb, gamma, beta, bsz):
    return pl.pallas_call(
        _ln_body,
        grid=(SEQ // BLK,),
        in_specs=[
            pl.BlockSpec((bsz, BLK, HIDDEN), lambda i: (0, i, 0)),
            pl.BlockSpec((BLK, HIDDEN), lambda i: (i, 0)),
            pl.BlockSpec((bsz, BLK, 1), lambda i: (0, i, 0)),
            pl.BlockSpec((2, HIDDEN), lambda i: (0, 0)),
            pl.BlockSpec((1, HIDDEN), lambda i: (0, 0)),
            pl.BlockSpec((1, HIDDEN), lambda i: (0, 0)),
        ],
        out_specs=pl.BlockSpec((bsz, BLK, HIDDEN), lambda i: (0, i, 0)),
        out_shape=jax.ShapeDtypeStruct((bsz, SEQ, HIDDEN), jnp.float32),
    )(gathered, pos_emb, ttf, type_emb, gamma, beta)


@jax.jit
def kernel(input_ids, token_type_ids, word_emb, pos_emb, type_emb, gamma, beta):
    bsz, seq = input_ids.shape
    ids = input_ids.reshape(-1).astype(jnp.int32)
    ttf = token_type_ids.reshape(bsz, seq, 1).astype(jnp.float32)
    gathered = _sc_gather(ids, word_emb).reshape(bsz, seq, HIDDEN)
    out = _tc_layernorm(gathered, pos_emb, ttf, type_emb,
                        gamma.reshape(1, HIDDEN), beta.reshape(1, HIDDEN), bsz)
    return out
